# baseline (speedup 1.0000x reference)
import jax
import jax.numpy as jnp
from jax import lax
from jax.experimental import pallas as pl
from jax.experimental.pallas import tpu as pltpu


def _body(idx_ref, we_ref, wut_ref, out_ref, emb_ref, sem):
    S = emb_ref.shape[0]

    @pl.when(pl.program_id(0) == 0)
    def _gather():
        Q = S // 4

        def issue(i, _):
            row = idx_ref[0, i]
            pltpu.make_async_copy(
                we_ref.at[pl.ds(row, 1)],
                emb_ref.at[pl.ds(i, 1)],
                sem.at[i // Q],
            ).start()
            return _

        lax.fori_loop(0, S, issue, 0, unroll=16)
        for q in range(4):
            pltpu.make_async_copy(
                we_ref.at[pl.ds(q * Q, Q)],
                emb_ref.at[pl.ds(q * Q, Q)],
                sem.at[q],
            ).wait()

    out_ref[...] = lax.dot_general(
        wut_ref[...],
        emb_ref[...],
        (((1,), (1,)), ((), ())),
        precision=lax.Precision.DEFAULT,
        preferred_element_type=jnp.float32,
    )


def kernel(x, W_E, W_U):
    B, S = x.shape
    V, D = W_E.shape
    M = B * S
    v_blk = 2000
    out_t = pl.pallas_call(
        _body,
        grid=(V // v_blk,),
        in_specs=[
            pl.BlockSpec(memory_space=pltpu.SMEM),
            pl.BlockSpec(memory_space=pl.ANY),
            pl.BlockSpec((v_blk, D), lambda n: (n, 0)),
        ],
        out_specs=pl.BlockSpec((v_blk, M), lambda n: (n, 0)),
        out_shape=jax.ShapeDtypeStruct((V, M), jnp.float32),
        scratch_shapes=[
            pltpu.VMEM((M, D), jnp.float32),
            pltpu.SemaphoreType.DMA((4,)),
        ],
        compiler_params=pltpu.CompilerParams(
            dimension_semantics=("arbitrary",),
        ),
    )(x.astype(jnp.int32), W_E, W_U.T)
    return out_t.T.reshape(B, S, V)


# speedup vs baseline: 1.0255x; 1.0255x over previous
import jax
import jax.numpy as jnp
from jax import lax
from jax.experimental import pallas as pl
from jax.experimental.pallas import tpu as pltpu


def _body(idx_ref, we_ref, wut_ref, out_ref, emb_ref, sem):
    S = emb_ref.shape[0]

    @pl.when(pl.program_id(0) == 0)
    def _gather():
        def issue(i, _):
            row = idx_ref[0, i]
            pltpu.make_async_copy(
                we_ref.at[pl.ds(row, 1)], emb_ref.at[pl.ds(i, 1)], sem
            ).start()
            return _

        lax.fori_loop(0, S, issue, 0, unroll=16)
        pltpu.make_async_copy(we_ref.at[pl.ds(0, S)], emb_ref, sem).wait()

    out_ref[...] = lax.dot_general(
        wut_ref[...],
        emb_ref[...],
        (((1,), (1,)), ((), ())),
        precision=lax.Precision.DEFAULT,
        preferred_element_type=jnp.float32,
    )


def kernel(x, W_E, W_U):
    B, S = x.shape
    V, D = W_E.shape
    M = B * S
    v_blk = 2000
    out_t = pl.pallas_call(
        _body,
        grid=(V // v_blk,),
        in_specs=[
            pl.BlockSpec(memory_space=pltpu.SMEM),
            pl.BlockSpec(memory_space=pl.ANY),
            pl.BlockSpec((v_blk, D), lambda n: (n, 0)),
        ],
        out_specs=pl.BlockSpec((v_blk, M), lambda n: (n, 0)),
        out_shape=jax.ShapeDtypeStruct((V, M), jnp.float32),
        scratch_shapes=[
            pltpu.VMEM((M, D), jnp.float32),
            pltpu.SemaphoreType.DMA,
        ],
        compiler_params=pltpu.CompilerParams(
            dimension_semantics=("arbitrary",),
        ),
    )(x.astype(jnp.int32), W_E, W_U.T)
    return out_t.T.reshape(B, S, V)
